# baseline (speedup 1.0000x reference)
import functools

import jax
import jax.numpy as jnp
from jax import lax
from jax.experimental import pallas as pl
from jax.experimental.pallas import tpu as pltpu
from jax.experimental.pallas import tpu_sc as plsc

NC = 2
NS = 16
NW = NC * NS
LANE = 128


def _node_gather_kernel(u_pad, i_pad, node_pad):
    mesh = plsc.VectorSubcoreMesh(core_axis_name="c", subcore_axis_name="s",
                                  num_cores=NC, num_subcores=NS)
    n_user_chunks = u_pad // LANE
    n_item_chunks = i_pad // LANE

    @functools.partial(
        pl.kernel,
        out_type=jax.ShapeDtypeStruct((node_pad, 128), jnp.float32),
        mesh=mesh,
        scratch_types=[
            pltpu.VMEM((LANE,), jnp.int32),
            pltpu.VMEM((LANE, 128), jnp.float32),
            pltpu.SemaphoreType.DMA,
        ],
    )
    def k(user_hbm, item_hbm, uidx_hbm, iidx_hbm, node_out, idx_v, rows_v, sem):
        c = lax.axis_index("c")
        s = lax.axis_index("s")
        w = c * NS + s

        def gather_chunk(table_hbm, idx1_hbm, chunk, row_base):
            pltpu.sync_copy(idx1_hbm.at[pl.ds(chunk * LANE, LANE)], idx_v)
            pltpu.async_copy(table_hbm.at[idx_v], rows_v, sem).wait()
            pltpu.sync_copy(rows_v, node_out.at[pl.ds(row_base, LANE)])

        for t in range(-(-n_user_chunks // NW)):
            chunk = t * NW + w
            if (t + 1) * NW <= n_user_chunks:
                gather_chunk(user_hbm, uidx_hbm, chunk, chunk * LANE)
            else:
                @pl.when(chunk < n_user_chunks)
                def _():
                    gather_chunk(user_hbm, uidx_hbm, chunk, chunk * LANE)
        for t in range(-(-n_item_chunks // NW)):
            chunk = t * NW + w
            if (t + 1) * NW <= n_item_chunks:
                gather_chunk(item_hbm, iidx_hbm, chunk, u_pad + chunk * LANE)
            else:
                @pl.when(chunk < n_item_chunks)
                def _():
                    gather_chunk(item_hbm, iidx_hbm, chunk, u_pad + chunk * LANE)

    return k


NBUF = 2


def _edge_kernel(node_pad, n_chunks):
    mesh = plsc.VectorSubcoreMesh(core_axis_name="c", subcore_axis_name="s",
                                  num_cores=NC, num_subcores=NS)
    rows_per_sub = node_pad // NS
    assert n_chunks % NBUF == 0

    @functools.partial(
        pl.kernel,
        out_type=[
            jax.ShapeDtypeStruct((NC, node_pad, 128), jnp.float32),
            jax.ShapeDtypeStruct((NC, NS, 1, rows_per_sub), jnp.float32),
        ],
        mesh=mesh,
        scratch_types=[
            pltpu.VMEM((n_chunks, LANE), jnp.int32),
            pltpu.VMEM((NBUF, LANE), jnp.int32),
            pltpu.VMEM((NBUF, LANE, 128), jnp.float32),
            pltpu.VMEM((LANE,), jnp.float32),
            pltpu.VMEM_SHARED((node_pad, 128), jnp.float32),
            pltpu.VMEM_SHARED((node_pad,), jnp.float32),
            [pltpu.SemaphoreType.DMA] * NBUF,
            [pltpu.SemaphoreType.DMA] * NBUF,
            [pltpu.SemaphoreType.DMA] * NBUF,
            [pltpu.SemaphoreType.DMA] * NBUF,
        ],
    )
    def k(node_hbm, src_hbm, dst_hbm, ones_hbm, zrows_hbm, zdeg_hbm,
          agg_out, deg_out, src_v, dstr_v, rows_v, ones_v, agg_sh, deg_sh,
          gsem, ssem, dsem, isem):
        c = lax.axis_index("c")
        s = lax.axis_index("s")
        w = c * NS + s
        pltpu.sync_copy(zrows_hbm, agg_sh.at[pl.ds(s * rows_per_sub, rows_per_sub)])
        pltpu.sync_copy(zdeg_hbm, deg_sh.at[pl.ds(s * rows_per_sub, rows_per_sub)])
        pltpu.sync_copy(src_hbm.at[pl.ds(w * n_chunks, n_chunks)], src_v)
        pltpu.sync_copy(ones_hbm, ones_v)

        def gather_start(jj, b):
            pltpu.async_copy(node_hbm.at[src_v.at[jj]], rows_v.at[b], gsem[b])

        def gather_wait(jj, b):
            pltpu.make_async_copy(node_hbm.at[src_v.at[jj]], rows_v.at[b],
                                  gsem[b]).wait()

        def didx_start(jj, b):
            pltpu.async_copy(dst_hbm.at[w * n_chunks + jj], dstr_v.at[b],
                             isem[b])

        def didx_wait(jj, b):
            pltpu.make_async_copy(dst_hbm.at[w * n_chunks + jj], dstr_v.at[b],
                                  isem[b]).wait()

        for b in range(NBUF):
            didx_start(b, b)
            gather_start(b, b)
        plsc.subcore_barrier()

        def body(i, carry):
            j = i * NBUF
            sdescs = []
            ddescs = []
            for b in range(NBUF):
                jj = j + b
                gather_wait(jj, b)
                didx_wait(jj, b)
                sdescs.append(pltpu.async_copy(
                    rows_v.at[b], agg_sh.at[dstr_v.at[b]], ssem[b], add=True))
                ddescs.append(pltpu.async_copy(
                    ones_v, deg_sh.at[dstr_v.at[b]], dsem[b], add=True))
            for b in range(NBUF):
                sdescs[b].wait()
                ddescs[b].wait()
                didx_start(j + NBUF + b, b)
                gather_start(j + NBUF + b, b)
            return carry

        lax.fori_loop(0, n_chunks // NBUF - 1, body, 0)
        for b in range(NBUF):
            jj = n_chunks - NBUF + b
            gather_wait(jj, b)
            didx_wait(jj, b)
            pltpu.sync_copy(rows_v.at[b], agg_sh.at[dstr_v.at[b]], add=True)
            pltpu.sync_copy(ones_v, deg_sh.at[dstr_v.at[b]], add=True)
        plsc.subcore_barrier()
        sl = pl.ds(s * rows_per_sub, rows_per_sub)
        pltpu.sync_copy(agg_sh.at[sl], agg_out.at[c, sl])
        pltpu.sync_copy(deg_sh.at[sl], deg_out.at[c, s, 0])

    return k


def _tc_self_kernel(node_pad, blk=512):
    def body(node_ref, ws_ref, b_ref, out_ref):
        out_ref[...] = jnp.dot(node_ref[...], ws_ref[...],
                               preferred_element_type=jnp.float32) + b_ref[...]

    return pl.pallas_call(
        body,
        grid=(node_pad // blk,),
        in_specs=[
            pl.BlockSpec((blk, 128), lambda i: (i, 0)),
            pl.BlockSpec((128, 128), lambda i: (0, 0)),
            pl.BlockSpec((1, 128), lambda i: (0, 0)),
        ],
        out_specs=pl.BlockSpec((blk, 128), lambda i: (i, 0)),
        out_shape=jax.ShapeDtypeStruct((node_pad, 128), jnp.float32),
    )


def _tc_combine_kernel(node_pad, blk=512):
    def body(hself_ref, agg_ref, deg_ref, wn_ref, out_ref):
        a = agg_ref[0] + agg_ref[1]
        d = deg_ref[:, 0:1] + deg_ref[:, 1:2]
        agg = a / jnp.maximum(d, 1.0)
        h = hself_ref[...] + jnp.dot(agg, wn_ref[...],
                                     preferred_element_type=jnp.float32)
        out_ref[...] = jax.nn.sigmoid(h)

    grid = (node_pad // blk,)
    return pl.pallas_call(
        body,
        grid=grid,
        in_specs=[
            pl.BlockSpec((blk, 128), lambda i: (i, 0)),
            pl.BlockSpec((2, blk, 128), lambda i: (0, i, 0)),
            pl.BlockSpec((blk, 2), lambda i: (i, 0)),
            pl.BlockSpec((128, 128), lambda i: (0, 0)),
        ],
        out_specs=pl.BlockSpec((blk, 128), lambda i: (i, 0)),
        out_shape=jax.ShapeDtypeStruct((node_pad, 128), jnp.float32),
    )


def kernel(user_feats, item_feats, user_order_in_graph, item_order_in_graph,
           edge_index, W_self, W_neigh, b):
    nu = user_order_in_graph.shape[0]
    ni = item_order_in_graph.shape[0]
    e = edge_index.shape[1]

    u_pad = -(-nu // LANE) * LANE
    i_pad = -(-ni // LANE) * LANE
    node_pad = -(-(u_pad + i_pad) // (NS * LANE)) * (NS * LANE)

    uidx1 = jnp.pad(user_order_in_graph.astype(jnp.int32), (0, u_pad - nu))
    iidx1 = jnp.pad(item_order_in_graph.astype(jnp.int32), (0, i_pad - ni))

    src = edge_index[0].astype(jnp.int32)
    dst = edge_index[1].astype(jnp.int32)
    shift = u_pad - nu
    srcp = jnp.where(src < nu, src, src + shift)
    dstp = jnp.where(dst < nu, dst, dst + shift)
    epw = e // NW
    n_chunks = -(-epw // LANE)
    n_chunks = -(-n_chunks // 8) * 8
    epw_pad = n_chunks * LANE
    srcp2 = jnp.pad(srcp.reshape(NW, epw), ((0, 0), (0, epw_pad - epw))
                    ).reshape(NW * n_chunks, LANE)
    dstp2 = jnp.pad(dstp.reshape(NW, epw), ((0, 0), (0, epw_pad - epw)),
                    constant_values=node_pad - 1).reshape(NW * n_chunks, LANE)

    node_feats = _node_gather_kernel(u_pad, i_pad, node_pad)(
        user_feats, item_feats, uidx1, iidx1)

    rows_per_sub = node_pad // NS
    ones128 = jnp.ones((LANE,), jnp.float32)
    zrows = jnp.zeros((rows_per_sub, 128), jnp.float32)
    zdeg = jnp.zeros((rows_per_sub,), jnp.float32)
    agg2, deg2 = _edge_kernel(node_pad, n_chunks)(
        node_feats, srcp2, dstp2, ones128, zrows, zdeg)
    hself = _tc_self_kernel(node_pad)(node_feats, W_self, b.reshape(1, 128))

    degT = deg2.reshape(NC, node_pad).T
    h = _tc_combine_kernel(node_pad)(hself, agg2, degT, W_neigh)

    return jnp.concatenate([h[:nu], h[u_pad:u_pad + ni]], axis=0)


# speedup vs baseline: 1.0041x; 1.0041x over previous
import functools

import jax
import jax.numpy as jnp
from jax import lax
from jax.experimental import pallas as pl
from jax.experimental.pallas import tpu as pltpu
from jax.experimental.pallas import tpu_sc as plsc

NC = 2
NS = 16
NW = NC * NS
LANE = 128


def _node_gather_kernel(u_pad, i_pad, node_pad):
    mesh = plsc.VectorSubcoreMesh(core_axis_name="c", subcore_axis_name="s",
                                  num_cores=NC, num_subcores=NS)
    n_user_chunks = u_pad // LANE
    n_item_chunks = i_pad // LANE

    @functools.partial(
        pl.kernel,
        out_type=jax.ShapeDtypeStruct((node_pad, 128), jnp.float32),
        mesh=mesh,
        scratch_types=[
            pltpu.VMEM((LANE,), jnp.int32),
            pltpu.VMEM((LANE, 128), jnp.float32),
            pltpu.SemaphoreType.DMA,
        ],
    )
    def k(user_hbm, item_hbm, uidx_hbm, iidx_hbm, node_out, idx_v, rows_v, sem):
        c = lax.axis_index("c")
        s = lax.axis_index("s")
        w = c * NS + s

        def gather_chunk(table_hbm, idx1_hbm, chunk, row_base):
            pltpu.sync_copy(idx1_hbm.at[pl.ds(chunk * LANE, LANE)], idx_v)
            pltpu.async_copy(table_hbm.at[idx_v], rows_v, sem).wait()
            pltpu.sync_copy(rows_v, node_out.at[pl.ds(row_base, LANE)])

        for t in range(-(-n_user_chunks // NW)):
            chunk = t * NW + w
            if (t + 1) * NW <= n_user_chunks:
                gather_chunk(user_hbm, uidx_hbm, chunk, chunk * LANE)
            else:
                @pl.when(chunk < n_user_chunks)
                def _():
                    gather_chunk(user_hbm, uidx_hbm, chunk, chunk * LANE)
        for t in range(-(-n_item_chunks // NW)):
            chunk = t * NW + w
            if (t + 1) * NW <= n_item_chunks:
                gather_chunk(item_hbm, iidx_hbm, chunk, u_pad + chunk * LANE)
            else:
                @pl.when(chunk < n_item_chunks)
                def _():
                    gather_chunk(item_hbm, iidx_hbm, chunk, u_pad + chunk * LANE)

    return k


NBUF = 2


def _edge_kernel(node_pad, n_chunks):
    mesh = plsc.VectorSubcoreMesh(core_axis_name="c", subcore_axis_name="s",
                                  num_cores=NC, num_subcores=NS)
    rows_per_sub = node_pad // NS
    assert n_chunks % NBUF == 0

    @functools.partial(
        pl.kernel,
        out_type=[
            jax.ShapeDtypeStruct((NC, node_pad, 128), jnp.float32),
            jax.ShapeDtypeStruct((NC, NS, 1, rows_per_sub), jnp.float32),
        ],
        mesh=mesh,
        scratch_types=[
            pltpu.VMEM((n_chunks, LANE), jnp.int32),
            pltpu.VMEM((NBUF, LANE), jnp.int32),
            pltpu.VMEM((NBUF, LANE, 128), jnp.float32),
            pltpu.VMEM((LANE,), jnp.float32),
            pltpu.VMEM_SHARED((node_pad, 128), jnp.float32),
            pltpu.VMEM_SHARED((node_pad,), jnp.float32),
            [pltpu.SemaphoreType.DMA] * NBUF,
            [pltpu.SemaphoreType.DMA] * NBUF,
            [pltpu.SemaphoreType.DMA] * NBUF,
            [pltpu.SemaphoreType.DMA] * NBUF,
        ],
    )
    def k(node_hbm, src_hbm, dst_hbm, ones_hbm, zrows_hbm, zdeg_hbm,
          agg_out, deg_out, src_v, dstr_v, rows_v, ones_v, agg_sh, deg_sh,
          gsem, ssem, dsem, isem):
        c = lax.axis_index("c")
        s = lax.axis_index("s")
        w = c * NS + s
        pltpu.sync_copy(zrows_hbm, agg_sh.at[pl.ds(s * rows_per_sub, rows_per_sub)])
        pltpu.sync_copy(zdeg_hbm, deg_sh.at[pl.ds(s * rows_per_sub, rows_per_sub)])
        pltpu.sync_copy(src_hbm.at[pl.ds(w * n_chunks, n_chunks)], src_v)
        pltpu.sync_copy(ones_hbm, ones_v)

        def gather_start(jj, b):
            pltpu.async_copy(node_hbm.at[src_v.at[jj]], rows_v.at[b], gsem[b])

        def gather_wait(jj, b):
            pltpu.make_async_copy(node_hbm.at[src_v.at[jj]], rows_v.at[b],
                                  gsem[b]).wait()

        def didx_start(jj, b):
            pltpu.async_copy(dst_hbm.at[w * n_chunks + jj], dstr_v.at[b],
                             isem[b])

        def didx_wait(jj, b):
            pltpu.make_async_copy(dst_hbm.at[w * n_chunks + jj], dstr_v.at[b],
                                  isem[b]).wait()

        for b in range(NBUF):
            didx_start(b, b)
            gather_start(b, b)
        plsc.subcore_barrier()

        def body(i, carry):
            j = i * NBUF
            sdescs = []
            ddescs = []
            for b in range(NBUF):
                jj = j + b
                gather_wait(jj, b)
                didx_wait(jj, b)
                sdescs.append(pltpu.async_copy(
                    rows_v.at[b], agg_sh.at[dstr_v.at[b]], ssem[b], add=True))
                ddescs.append(pltpu.async_copy(
                    ones_v, deg_sh.at[dstr_v.at[b]], dsem[b], add=True))
            for b in range(NBUF):
                sdescs[b].wait()
                ddescs[b].wait()
                didx_start(j + NBUF + b, b)
                gather_start(j + NBUF + b, b)
            return carry

        lax.fori_loop(0, n_chunks // NBUF - 1, body, 0)
        for b in range(NBUF):
            jj = n_chunks - NBUF + b
            gather_wait(jj, b)
            didx_wait(jj, b)
            pltpu.sync_copy(rows_v.at[b], agg_sh.at[dstr_v.at[b]], add=True)
            pltpu.sync_copy(ones_v, deg_sh.at[dstr_v.at[b]], add=True)
        plsc.subcore_barrier()
        sl = pl.ds(s * rows_per_sub, rows_per_sub)
        pltpu.sync_copy(agg_sh.at[sl], agg_out.at[c, sl])
        pltpu.sync_copy(deg_sh.at[sl], deg_out.at[c, s, 0])

    return k


def _tc_combine_kernel(node_pad, blk=512):
    def body(node_ref, agg_ref, deg_ref, ws_ref, wn_ref, b_ref, out_ref):
        a = agg_ref[0] + agg_ref[1]
        d = deg_ref[:, 0:1] + deg_ref[:, 1:2]
        agg = a / jnp.maximum(d, 1.0)
        h = (jnp.dot(node_ref[...], ws_ref[...], preferred_element_type=jnp.float32)
             + jnp.dot(agg, wn_ref[...], preferred_element_type=jnp.float32)
             + b_ref[...])
        out_ref[...] = jax.nn.sigmoid(h)

    grid = (node_pad // blk,)
    return pl.pallas_call(
        body,
        grid=grid,
        in_specs=[
            pl.BlockSpec((blk, 128), lambda i: (i, 0)),
            pl.BlockSpec((2, blk, 128), lambda i: (0, i, 0)),
            pl.BlockSpec((blk, 2), lambda i: (i, 0)),
            pl.BlockSpec((128, 128), lambda i: (0, 0)),
            pl.BlockSpec((128, 128), lambda i: (0, 0)),
            pl.BlockSpec((1, 128), lambda i: (0, 0)),
        ],
        out_specs=pl.BlockSpec((blk, 128), lambda i: (i, 0)),
        out_shape=jax.ShapeDtypeStruct((node_pad, 128), jnp.float32),
    )


def kernel(user_feats, item_feats, user_order_in_graph, item_order_in_graph,
           edge_index, W_self, W_neigh, b):
    nu = user_order_in_graph.shape[0]
    ni = item_order_in_graph.shape[0]
    e = edge_index.shape[1]

    u_pad = -(-nu // LANE) * LANE
    i_pad = -(-ni // LANE) * LANE
    node_pad = -(-(u_pad + i_pad) // (NS * LANE)) * (NS * LANE)

    uidx1 = jnp.pad(user_order_in_graph.astype(jnp.int32), (0, u_pad - nu))
    iidx1 = jnp.pad(item_order_in_graph.astype(jnp.int32), (0, i_pad - ni))

    src = edge_index[0].astype(jnp.int32)
    dst = edge_index[1].astype(jnp.int32)
    shift = u_pad - nu
    srcp = jnp.where(src < nu, src, src + shift)
    dstp = jnp.where(dst < nu, dst, dst + shift)
    epw = e // NW
    n_chunks = -(-epw // LANE)
    n_chunks = -(-n_chunks // 8) * 8
    epw_pad = n_chunks * LANE
    srcp2 = jnp.pad(srcp.reshape(NW, epw), ((0, 0), (0, epw_pad - epw))
                    ).reshape(NW * n_chunks, LANE)
    dstp2 = jnp.pad(dstp.reshape(NW, epw), ((0, 0), (0, epw_pad - epw)),
                    constant_values=node_pad - 1).reshape(NW * n_chunks, LANE)

    node_feats = _node_gather_kernel(u_pad, i_pad, node_pad)(
        user_feats, item_feats, uidx1, iidx1)

    rows_per_sub = node_pad // NS
    ones128 = jnp.ones((LANE,), jnp.float32)
    zrows = jnp.zeros((rows_per_sub, 128), jnp.float32)
    zdeg = jnp.zeros((rows_per_sub,), jnp.float32)
    agg2, deg2 = _edge_kernel(node_pad, n_chunks)(
        node_feats, srcp2, dstp2, ones128, zrows, zdeg)

    degT = deg2.reshape(NC, node_pad).T
    h = _tc_combine_kernel(node_pad)(
        node_feats, agg2, degT, W_self, W_neigh, b.reshape(1, 128))

    return jnp.concatenate([h[:nu], h[u_pad:u_pad + ni]], axis=0)
